# baseline (speedup 1.0000x reference)
import jax
import jax.numpy as jnp
from jax import lax
from jax.experimental import pallas as pl
from jax.experimental.pallas import tpu as pltpu

N_ROWS = 64
N_COLS = 32768
ROW_BLOCK = 8
GRID = N_ROWS // ROW_BLOCK
SUB = N_COLS // 128
K = 8
ITERS = SUB // K


def _argmin_body(x_ref, out_ref):
    def body(it, carry):
        acc = list(carry)
        for c in range(K):
            rm, ra = acc[2 * c], acc[2 * c + 1]
            sid = it * K + c
            vs = x_ref[:, pl.ds(sid * 128, 128)]
            upd = vs < rm
            acc[2 * c] = jnp.where(upd, vs, rm)
            acc[2 * c + 1] = jnp.where(upd, jnp.full(
                (ROW_BLOCK, 128), sid, jnp.int32), ra)
        return tuple(acc)

    init = []
    for _ in range(K):
        init += [jnp.full((ROW_BLOCK, 128), jnp.inf, jnp.float32),
                 jnp.zeros((ROW_BLOCK, 128), jnp.int32)]
    carry = lax.fori_loop(0, ITERS, body, tuple(init), unroll=True)

    mv, mi = carry[0], carry[1]
    for c in range(1, K):
        bv, bi = carry[2 * c], carry[2 * c + 1]
        upd = (bv < mv) | ((bv == mv) & (bi < mi))
        mv = jnp.where(upd, bv, mv)
        mi = jnp.where(upd, bi, mi)
    col = mi * 128 + lax.broadcasted_iota(jnp.int32, (ROW_BLOCK, 128), 1)
    m = jnp.min(mv, axis=1, keepdims=True)
    out_ref[...] = jnp.min(
        jnp.where(mv == m, col, jnp.int32(2**30)),
        axis=1, keepdims=True)


_argmin = pl.pallas_call(
    _argmin_body,
    grid=(GRID,),
    in_specs=[pl.BlockSpec((ROW_BLOCK, N_COLS), lambda i: (i, 0))],
    out_specs=pl.BlockSpec((ROW_BLOCK, 1), lambda i: (i, 0)),
    out_shape=jax.ShapeDtypeStruct((N_ROWS, 1), jnp.int32),
)


def kernel(x):
    return _argmin(x)[:, 0]


# speedup vs baseline: 1.4490x; 1.4490x over previous
import jax
import jax.numpy as jnp
from jax import lax
from jax.experimental import pallas as pl
from jax.experimental.pallas import tpu as pltpu

N_ROWS = 64
N_COLS = 32768
ROW_BLOCK = 8
CHUNKS = N_ROWS // ROW_BLOCK
SUB = N_COLS // 128
K = 8
ITERS = SUB // K


def _chunk_argmin(buf, ch):
    def body(it, carry):
        acc = list(carry)
        for c in range(K):
            rm, ra = acc[2 * c], acc[2 * c + 1]
            sid = it * K + c
            vs = buf[ch * ROW_BLOCK:(ch + 1) * ROW_BLOCK,
                     pl.ds(sid * 128, 128)]
            upd = vs < rm
            acc[2 * c] = jnp.where(upd, vs, rm)
            acc[2 * c + 1] = jnp.where(upd, jnp.full(
                (ROW_BLOCK, 128), sid, jnp.int32), ra)
        return tuple(acc)

    init = []
    for _ in range(K):
        init += [jnp.full((ROW_BLOCK, 128), jnp.inf, jnp.float32),
                 jnp.zeros((ROW_BLOCK, 128), jnp.int32)]
    carry = lax.fori_loop(0, ITERS, body, tuple(init), unroll=True)

    mv, mi = carry[0], carry[1]
    for c in range(1, K):
        bv, bi = carry[2 * c], carry[2 * c + 1]
        upd = (bv < mv) | ((bv == mv) & (bi < mi))
        mv = jnp.where(upd, bv, mv)
        mi = jnp.where(upd, bi, mi)
    col = mi * 128 + lax.broadcasted_iota(jnp.int32, (ROW_BLOCK, 128), 1)
    m = jnp.min(mv, axis=1, keepdims=True)
    return jnp.min(jnp.where(mv == m, col, jnp.int32(2**30)),
                   axis=1, keepdims=True)


def _argmin_body(x_hbm, out_ref, buf, sems):
    for ch in range(CHUNKS):
        pltpu.make_async_copy(
            x_hbm.at[pl.ds(ch * ROW_BLOCK, ROW_BLOCK)],
            buf.at[pl.ds(ch * ROW_BLOCK, ROW_BLOCK)],
            sems.at[ch],
        ).start()
    for ch in range(CHUNKS):
        pltpu.make_async_copy(
            x_hbm.at[pl.ds(ch * ROW_BLOCK, ROW_BLOCK)],
            buf.at[pl.ds(ch * ROW_BLOCK, ROW_BLOCK)],
            sems.at[ch],
        ).wait()
        out_ref[ch * ROW_BLOCK:(ch + 1) * ROW_BLOCK, :] = _chunk_argmin(
            buf, ch)


_argmin = pl.pallas_call(
    _argmin_body,
    in_specs=[pl.BlockSpec(memory_space=pl.ANY)],
    out_shape=jax.ShapeDtypeStruct((N_ROWS, 1), jnp.int32),
    scratch_shapes=[
        pltpu.VMEM((N_ROWS, N_COLS), jnp.float32),
        pltpu.SemaphoreType.DMA((CHUNKS,)),
    ],
)


def kernel(x):
    return _argmin(x)[:, 0]
